# baseline (speedup 1.0000x reference)
import functools

import jax
import jax.numpy as jnp
from jax import lax
from jax.experimental import pallas as pl
from jax.experimental.pallas import tpu as pltpu
from jax.experimental.pallas import tpu_sc as plsc

_NC = 2
_NS = 16
_NW = _NC * _NS
_CBQ = 8192


def _pack_s(V):
    return ((V + 8 * _CBQ - 1) // (8 * _CBQ)) * _CBQ


def _pack_body(*refs):
    out_ref = refs[-1]
    parts = []
    for g in range(4):
        lo = jnp.transpose(refs[g][...], (1, 0)).astype(jnp.bfloat16)
        hi = jnp.transpose(refs[g + 4][...], (1, 0)).astype(jnp.bfloat16)
        lo32 = lax.bitcast_convert_type(lo, jnp.uint16).astype(jnp.uint32)
        hi32 = lax.bitcast_convert_type(hi, jnp.uint16).astype(jnp.uint32)
        parts.append(lo32 | (hi32 << 16))
    out_ref[...] = jnp.concatenate(parts, axis=1)


@jax.jit
def _pack(tt):
    V = tt.shape[1]
    nblk = _pack_s(V) // _CBQ
    last = (V - 1) // _CBQ
    packed = pl.pallas_call(
        _pack_body,
        grid=(nblk,),
        in_specs=[
            pl.BlockSpec(
                (32, _CBQ),
                lambda i, k=k: (0, jnp.minimum(k * nblk + i, last)),
            )
            for k in range(8)
        ],
        out_specs=pl.BlockSpec((_CBQ, 128), lambda i: (i, 0)),
        out_shape=jax.ShapeDtypeStruct((nblk * _CBQ, 128), jnp.uint32),
    )(*([tt] * 8))
    return packed, nblk * _CBQ


@functools.partial(jax.jit, static_argnums=(0,))
def _sc_gather(b_per_w, packed, q):
    B = b_per_w * _NW
    mesh = plsc.VectorSubcoreMesh(core_axis_name="c", subcore_axis_name="s")

    @functools.partial(
        pl.kernel,
        mesh=mesh,
        out_type=jax.ShapeDtypeStruct((B, 128), jnp.uint32),
        scratch_types=[
            pltpu.VMEM((b_per_w,), jnp.int32),
            pltpu.VMEM((b_per_w, 128), jnp.uint32),
            pltpu.SemaphoreType.DMA,
        ],
        compiler_params=pltpu.CompilerParams(use_tc_tiling_on_sc=True),
    )
    def gather_k(p_hbm, q_hbm, out_hbm, q_v, row_v, sem):
        wid = lax.axis_index("s") * _NC + lax.axis_index("c")
        base = wid * b_per_w
        pltpu.sync_copy(q_hbm.at[pl.ds(base, b_per_w)], q_v)
        pltpu.async_copy(p_hbm.at[q_v], row_v, sem).wait()
        pltpu.sync_copy(row_v, out_hbm.at[pl.ds(base, b_per_w)])

    return gather_k(packed, q)


def _mlp_body(uep_ref, mep_ref, su_ref, sm_ref, w1a_ref, w1b_ref, b1_ref,
              w2_ref, b2_ref, w3_ref, b3_ref, out_ref):
    uep = uep_ref[...]
    mep = mep_ref[...]
    su = su_ref[...]
    sm = sm_ref[...]
    blk = uep.shape[0]
    lane_group = lax.broadcasted_iota(jnp.int32, (blk, 128), 1) // 32

    def extract(packed, sel):
        lo = lax.bitcast_convert_type(packed << 16, jnp.float32)
        hi = lax.bitcast_convert_type(packed & jnp.uint32(0xFFFF0000),
                                      jnp.float32)
        val = jnp.where(sel >= 4, hi, lo)
        return jnp.where(lane_group == sel % 4, val, 0.0)

    ue = extract(uep, su)
    me = extract(mep, sm)
    h = (
        jnp.dot(ue, w1a_ref[...], preferred_element_type=jnp.float32)
        + jnp.dot(me, w1b_ref[...], preferred_element_type=jnp.float32)
        + b1_ref[...]
    )
    h = jnp.maximum(h, 0.0)
    h = jnp.dot(h, w2_ref[...], preferred_element_type=jnp.float32) + b2_ref[...]
    h = jnp.maximum(h, 0.0)
    z = jnp.sum(h * w3_ref[...], axis=1, keepdims=True) + b3_ref[...]
    out_ref[...] = 5.0 / (1.0 + jnp.exp(-z))


@functools.partial(jax.jit, static_argnums=(0,))
def _tc_mlp(blk, uep, mep, su, sm, W1a, W1b, b1, W2, b2, W3r, b3):
    B = uep.shape[0]
    grid = (B // blk,)
    return pl.pallas_call(
        _mlp_body,
        grid=grid,
        in_specs=[
            pl.BlockSpec((blk, 128), lambda i: (i, 0)),
            pl.BlockSpec((blk, 128), lambda i: (i, 0)),
            pl.BlockSpec((blk, 1), lambda i: (i, 0)),
            pl.BlockSpec((blk, 1), lambda i: (i, 0)),
            pl.BlockSpec((128, 256), lambda i: (0, 0)),
            pl.BlockSpec((128, 256), lambda i: (0, 0)),
            pl.BlockSpec((1, 256), lambda i: (0, 0)),
            pl.BlockSpec((256, 64), lambda i: (0, 0)),
            pl.BlockSpec((1, 64), lambda i: (0, 0)),
            pl.BlockSpec((1, 64), lambda i: (0, 0)),
            pl.BlockSpec((1, 1), lambda i: (0, 0)),
        ],
        out_specs=pl.BlockSpec((blk, 1), lambda i: (i, 0)),
        out_shape=jax.ShapeDtypeStruct((B, 1), jnp.float32),
    )(uep, mep, su, sm, W1a, W1b, b1, W2, b2, W3r, b3)


def kernel(userId, movieId, user_table, movie_table, W1, b1, W2, b2, W3, b3):
    B = userId.shape[0]
    D = user_table.shape[1]
    uid = userId.astype(jnp.int32)
    mid = movieId.astype(jnp.int32)
    su = (uid // _pack_s(user_table.shape[0])).reshape(B, 1)
    sm = (mid // _pack_s(movie_table.shape[0])).reshape(B, 1)
    movie_packed, sm_ = _pack(movie_table.T)
    me_p = _sc_gather(B // _NW, movie_packed, mid % sm_)
    user_packed, su_ = _pack(user_table.T)
    ue_p = _sc_gather(B // _NW, user_packed, uid % su_)
    w1a4 = jnp.concatenate([W1[:D]] * 4, axis=0)
    w1b4 = jnp.concatenate([W1[D:]] * 4, axis=0)
    out = _tc_mlp(
        4096, ue_p, me_p, su, sm,
        w1a4, w1b4, b1.reshape(1, -1), W2, b2.reshape(1, -1),
        W3.reshape(1, -1), b3.reshape(1, 1),
    )
    return out


# speedup vs baseline: 1.0215x; 1.0215x over previous
import functools

import jax
import jax.numpy as jnp
from jax import lax
from jax.experimental import pallas as pl
from jax.experimental.pallas import tpu as pltpu
from jax.experimental.pallas import tpu_sc as plsc

_NC = 2
_NS = 16
_NW = _NC * _NS
_CBQ = 4096


def _pack_s(V):
    return ((V + 8 * _CBQ - 1) // (8 * _CBQ)) * _CBQ


def _pack_body(*refs):
    out_ref = refs[-1]
    parts = []
    for g in range(4):
        lo = jnp.transpose(refs[g][...], (1, 0)).astype(jnp.bfloat16)
        hi = jnp.transpose(refs[g + 4][...], (1, 0)).astype(jnp.bfloat16)
        lo32 = lax.bitcast_convert_type(lo, jnp.uint16).astype(jnp.uint32)
        hi32 = lax.bitcast_convert_type(hi, jnp.uint16).astype(jnp.uint32)
        parts.append(lo32 | (hi32 << 16))
    out_ref[...] = jnp.concatenate(parts, axis=1)


@jax.jit
def _pack(tt):
    V = tt.shape[1]
    nblk = _pack_s(V) // _CBQ
    last = (V - 1) // _CBQ
    packed = pl.pallas_call(
        _pack_body,
        grid=(nblk,),
        in_specs=[
            pl.BlockSpec(
                (32, _CBQ),
                lambda i, k=k: (0, jnp.minimum(k * nblk + i, last)),
            )
            for k in range(8)
        ],
        out_specs=pl.BlockSpec((_CBQ, 128), lambda i: (i, 0)),
        out_shape=jax.ShapeDtypeStruct((nblk * _CBQ, 128), jnp.uint32),
    )(*([tt] * 8))
    return packed, nblk * _CBQ


@functools.partial(jax.jit, static_argnums=(0,))
def _sc_gather(b_per_w, packed, q):
    B = b_per_w * _NW
    mesh = plsc.VectorSubcoreMesh(core_axis_name="c", subcore_axis_name="s")

    @functools.partial(
        pl.kernel,
        mesh=mesh,
        out_type=jax.ShapeDtypeStruct((B, 128), jnp.uint32),
        scratch_types=[
            pltpu.VMEM((b_per_w,), jnp.int32),
            pltpu.VMEM((b_per_w, 128), jnp.uint32),
            pltpu.SemaphoreType.DMA,
        ],
        compiler_params=pltpu.CompilerParams(use_tc_tiling_on_sc=True),
    )
    def gather_k(p_hbm, q_hbm, out_hbm, q_v, row_v, sem):
        wid = lax.axis_index("s") * _NC + lax.axis_index("c")
        base = wid * b_per_w
        pltpu.sync_copy(q_hbm.at[pl.ds(base, b_per_w)], q_v)
        pltpu.async_copy(p_hbm.at[q_v], row_v, sem).wait()
        pltpu.sync_copy(row_v, out_hbm.at[pl.ds(base, b_per_w)])

    return gather_k(packed, q)


def _mlp_body(uep_ref, mep_ref, su_ref, sm_ref, w1a_ref, w1b_ref, b1_ref,
              w2_ref, b2_ref, w3_ref, b3_ref, out_ref):
    uep = uep_ref[...]
    mep = mep_ref[...]
    su = su_ref[...]
    sm = sm_ref[...]
    blk = uep.shape[0]
    lane_group = lax.broadcasted_iota(jnp.int32, (blk, 128), 1) // 32

    def extract(packed, sel):
        lo = lax.bitcast_convert_type(packed << 16, jnp.float32)
        hi = lax.bitcast_convert_type(packed & jnp.uint32(0xFFFF0000),
                                      jnp.float32)
        val = jnp.where(sel >= 4, hi, lo)
        return jnp.where(lane_group == sel % 4, val, 0.0)

    ue = extract(uep, su)
    me = extract(mep, sm)
    h = (
        jnp.dot(ue, w1a_ref[...], preferred_element_type=jnp.float32)
        + jnp.dot(me, w1b_ref[...], preferred_element_type=jnp.float32)
        + b1_ref[...]
    )
    h = jnp.maximum(h, 0.0)
    h = jnp.dot(h, w2_ref[...], preferred_element_type=jnp.float32) + b2_ref[...]
    h = jnp.maximum(h, 0.0)
    z = jnp.sum(h * w3_ref[...], axis=1, keepdims=True) + b3_ref[...]
    out_ref[...] = 5.0 / (1.0 + jnp.exp(-z))


@functools.partial(jax.jit, static_argnums=(0,))
def _tc_mlp(blk, uep, mep, su, sm, W1a, W1b, b1, W2, b2, W3r, b3):
    B = uep.shape[0]
    grid = (B // blk,)
    return pl.pallas_call(
        _mlp_body,
        grid=grid,
        in_specs=[
            pl.BlockSpec((blk, 128), lambda i: (i, 0)),
            pl.BlockSpec((blk, 128), lambda i: (i, 0)),
            pl.BlockSpec((blk, 1), lambda i: (i, 0)),
            pl.BlockSpec((blk, 1), lambda i: (i, 0)),
            pl.BlockSpec((128, 256), lambda i: (0, 0)),
            pl.BlockSpec((128, 256), lambda i: (0, 0)),
            pl.BlockSpec((1, 256), lambda i: (0, 0)),
            pl.BlockSpec((256, 64), lambda i: (0, 0)),
            pl.BlockSpec((1, 64), lambda i: (0, 0)),
            pl.BlockSpec((1, 64), lambda i: (0, 0)),
            pl.BlockSpec((1, 1), lambda i: (0, 0)),
        ],
        out_specs=pl.BlockSpec((blk, 1), lambda i: (i, 0)),
        out_shape=jax.ShapeDtypeStruct((B, 1), jnp.float32),
    )(uep, mep, su, sm, W1a, W1b, b1, W2, b2, W3r, b3)


def kernel(userId, movieId, user_table, movie_table, W1, b1, W2, b2, W3, b3):
    B = userId.shape[0]
    D = user_table.shape[1]
    uid = userId.astype(jnp.int32)
    mid = movieId.astype(jnp.int32)
    su = (uid // _pack_s(user_table.shape[0])).reshape(B, 1)
    sm = (mid // _pack_s(movie_table.shape[0])).reshape(B, 1)
    movie_packed, sm_ = _pack(movie_table.T)
    me_p = _sc_gather(B // _NW, movie_packed, mid % sm_)
    user_packed, su_ = _pack(user_table.T)
    ue_p = _sc_gather(B // _NW, user_packed, uid % su_)
    w1a4 = jnp.concatenate([W1[:D]] * 4, axis=0)
    w1b4 = jnp.concatenate([W1[D:]] * 4, axis=0)
    out = _tc_mlp(
        4096, ue_p, me_p, su, sm,
        w1a4, w1b4, b1.reshape(1, -1), W2, b2.reshape(1, -1),
        W3.reshape(1, -1), b3.reshape(1, 1),
    )
    return out
